# baseline (speedup 1.0000x reference)
import functools

import jax
import jax.numpy as jnp
from jax import lax
from jax.experimental import pallas as pl
from jax.experimental.pallas import tpu as pltpu
from jax.experimental.pallas import tpu_sc as plsc

_N = 10000
_E = 320000
_DIN = 128
_DH = 256
_DZ = 128
_G = 64
_W = 128

_NC = 2
_NS = 16
_CHUNK = 128
_CPT = 80
_NCH = _NC * _NS * _CPT
_EP = _NCH * _CHUNK
_CPT_COL = _NCH // _NS
_IB = 40
_RPT = 632
_NP = _NS * _RPT

_mesh = plsc.VectorSubcoreMesh(
    core_axis_name="c", subcore_axis_name="s", num_cores=_NC, num_subcores=_NS
)


def _deg_body(dst_hbm, ones_hbm, zeros_hbm, out_hbm, dst_v, ones_v, acc, sem):
    c = lax.axis_index("c")
    s = lax.axis_index("s")
    w = c * _NS + s
    pltpu.sync_copy(ones_hbm, ones_v)
    pltpu.sync_copy(dst_hbm.at[pl.ds(w * _CPT, _CPT)], dst_v)
    pltpu.sync_copy(zeros_hbm, acc.at[pl.ds(s * _RPT, _RPT)])
    plsc.subcore_barrier()

    pltpu.async_copy(ones_v, acc.at[dst_v.at[0]], sem, add=True)

    def body(i, carry):
        d = pltpu.async_copy(ones_v, acc.at[dst_v.at[i + 1]], sem, add=True)
        d.wait()
        return carry

    lax.fori_loop(0, _CPT - 1, body, 0)
    pltpu.make_async_copy(ones_hbm, ones_v, sem).wait()

    plsc.subcore_barrier()
    row0 = s * _RPT
    pltpu.sync_copy(
        acc.at[pl.ds(row0, _RPT)],
        out_hbm.at[pl.ds(c * _NP + row0, _RPT)],
    )


_deg_kernel = functools.partial(
    pl.kernel,
    _deg_body,
    out_type=jax.ShapeDtypeStruct((2 * _NP, _W), jnp.float32),
    mesh=_mesh,
    scratch_types=[
        pltpu.VMEM((_CPT, _CHUNK), jnp.int32),
        pltpu.VMEM((_CHUNK, _W), jnp.float32),
        pltpu.VMEM_SHARED((_NP, _W), jnp.float32),
        pltpu.SemaphoreType.DMA,
    ],
)()


def _edge_body(col_split, table_hbm, src_hbm, dst_hbm, zeros_hbm, out_hbm,
               src_v, dst_v, r0, r1, acc, g0, g1, s0, s1):
    c = lax.axis_index("c")
    s = lax.axis_index("s")
    if col_split:
        n = _CPT_COL
        base = s * n
        plane = c
    else:
        n = _CPT
        base = (c * _NS + s) * n
        plane = c
    pltpu.sync_copy(zeros_hbm, acc.at[pl.ds(s * _RPT, _RPT)])
    plsc.subcore_barrier()

    def wait_g0():
        pltpu.make_async_copy(table_hbm.at[src_v.at[0]], r0, g0).wait()

    def wait_g1():
        pltpu.make_async_copy(table_hbm.at[src_v.at[0]], r1, g1).wait()

    def wait_s0():
        pltpu.make_async_copy(table_hbm.at[pl.ds(0, _CHUNK)], r0, s0).wait()

    def wait_s1():
        pltpu.make_async_copy(table_hbm.at[pl.ds(0, _CHUNK)], r1, s1).wait()

    def pair(it, carry):
        i0 = 2 * it
        r = lax.rem(i0, _IB)

        @pl.when(r == 0)
        def _():
            @pl.when(i0 > 0)
            def _():
                wait_s1()

            off = pl.multiple_of(base + i0, 8)
            pltpu.sync_copy(src_hbm.at[plane, pl.ds(off, _IB)], src_v)
            pltpu.sync_copy(dst_hbm.at[pl.ds(off, _IB)], dst_v)
            pltpu.async_copy(table_hbm.at[src_v.at[0]], r0, g0)

        wait_g0()
        pltpu.async_copy(r0, acc.at[dst_v.at[r]], s0, add=True)

        @pl.when(r > 0)
        def _():
            wait_s1()

        pltpu.async_copy(table_hbm.at[src_v.at[r + 1]], r1, g1)
        wait_g1()
        pltpu.async_copy(r1, acc.at[dst_v.at[r + 1]], s1, add=True)
        wait_s0()

        @pl.when(r + 2 < _IB)
        def _():
            pltpu.async_copy(table_hbm.at[src_v.at[r + 2]], r0, g0)

        return carry

    lax.fori_loop(0, n // 2, pair, 0)
    wait_s1()

    plsc.subcore_barrier()
    row0 = s * _RPT
    pltpu.sync_copy(
        acc.at[pl.ds(row0, _RPT)],
        out_hbm.at[pl.ds(c * _NP + row0, _RPT)],
    )


def _make_edge_kernel(col_split):
    return functools.partial(
        pl.kernel,
        functools.partial(_edge_body, col_split),
        out_type=jax.ShapeDtypeStruct((2 * _NP, _W), jnp.float32),
        mesh=_mesh,
        scratch_types=[
            pltpu.VMEM((_IB, _CHUNK), jnp.int32),
            pltpu.VMEM((_IB, _CHUNK), jnp.int32),
            pltpu.VMEM((_CHUNK, _W), jnp.float32),
            pltpu.VMEM((_CHUNK, _W), jnp.float32),
            pltpu.VMEM_SHARED((_NP, _W), jnp.float32),
            pltpu.SemaphoreType.DMA,
            pltpu.SemaphoreType.DMA,
            pltpu.SemaphoreType.DMA,
            pltpu.SemaphoreType.DMA,
        ],
    )()


_edge_kernel_split = _make_edge_kernel(False)
_edge_kernel_cols = _make_edge_kernel(True)


def _elu(v):
    return jnp.where(v > 0, v, jnp.exp(jnp.minimum(v, 0.0)) - 1.0)


def _tc_prep_body(degp_ref, x_ref, dis_ref, x2_ref):
    deg = degp_ref[0:_N, 0:1] + degp_ref[_NP:_NP + _N, 0:1] + 1.0
    dis = lax.rsqrt(deg)
    dis_ref[...] = dis
    x2 = x_ref[...] * dis
    x2_ref[0:_N, :] = x2
    x2_ref[_NP:_NP + _N, :] = x2


def _tc_mid_body(s1_ref, x2_ref, dis_ref, w1_ref, b1_ref, h2s_ref):
    dis = dis_ref[...]
    ax = (s1_ref[0:_N, :] + s1_ref[_NP:_NP + _N, :] + x2_ref[0:_N, :]) * dis
    h = _elu(jnp.dot(ax, w1_ref[...], preferred_element_type=jnp.float32)
             + b1_ref[...])
    h2 = h * dis
    h2s_ref[0:_N, :] = h2[:, 0:_DH // 2]
    h2s_ref[_NP:_NP + _N, :] = h2[:, _DH // 2:_DH]


def _tc_head_body(s2_ref, h2s_ref, dis_ref, wmu_ref, bmu_ref, wsg_ref,
                  bsg_ref, batch_ref, zmu_ref, zsg_ref):
    dis = dis_ref[...]
    a = s2_ref[...] + h2s_ref[...]
    ah = jnp.concatenate([a[0:_N, :], a[_NP:_NP + _N, :]], axis=1) * dis
    mu = _elu(jnp.dot(ah, wmu_ref[...], preferred_element_type=jnp.float32)
              + bmu_ref[...])
    sg = _elu(jnp.dot(ah, wsg_ref[...], preferred_element_type=jnp.float32)
              + bsg_ref[...])
    gids = lax.broadcasted_iota(jnp.int32, (_G, _N), 0)
    p = (gids == batch_ref[...]).astype(jnp.float32)
    inv_cnt = 1.0 / jnp.maximum(jnp.sum(p, axis=1, keepdims=True), 1.0)
    zmu_ref[...] = jnp.dot(p, mu, preferred_element_type=jnp.float32) * inv_cnt
    zsg_ref[...] = jnp.dot(p, sg, preferred_element_type=jnp.float32) * inv_cnt


_tc_prep = pl.pallas_call(
    _tc_prep_body,
    out_shape=(
        jax.ShapeDtypeStruct((_N, 1), jnp.float32),
        jax.ShapeDtypeStruct((2 * _NP, _DIN), jnp.float32),
    ),
)

_tc_mid = pl.pallas_call(
    _tc_mid_body,
    out_shape=jax.ShapeDtypeStruct((2 * _NP, _DH // 2), jnp.float32),
)

_tc_head = pl.pallas_call(
    _tc_head_body,
    out_shape=(
        jax.ShapeDtypeStruct((_G, _DZ), jnp.float32),
        jax.ShapeDtypeStruct((_G, _DZ), jnp.float32),
    ),
)


def kernel(x, edge_index, batch, W1, b1, W_mu, b_mu, W_sigma, b_sigma):
    src = jnp.pad(edge_index[0], (0, _EP - _E), constant_values=_N)
    dst = jnp.pad(edge_index[1], (0, _EP - _E), constant_values=_N)
    src2 = src.reshape(_NCH, _CHUNK)
    src3 = jnp.stack([src2, src2 + _NP])
    dst2 = dst.reshape(_NCH, _CHUNK)
    ones_rows = jnp.ones((_CHUNK, _W), jnp.float32)
    zrows = jnp.zeros((_RPT, _W), jnp.float32)

    degp = _deg_kernel(dst2, ones_rows, zrows)
    dis, x2 = _tc_prep(degp, x)
    s1 = _edge_kernel_split(x2, src3, dst2, zrows)
    h2s = _tc_mid(s1, x2, dis, W1, b1.reshape(1, _DH))
    s2 = _edge_kernel_cols(h2s, src3, dst2, zrows)
    z_mu, z_sigma = _tc_head(
        s2, h2s, dis, W_mu, b_mu.reshape(1, _DZ),
        W_sigma, b_sigma.reshape(1, _DZ), batch.reshape(1, _N),
    )
    return (z_mu, z_sigma)


# speedup vs baseline: 1.0479x; 1.0479x over previous
import functools

import jax
import jax.numpy as jnp
from jax import lax
from jax.experimental import pallas as pl
from jax.experimental.pallas import tpu as pltpu
from jax.experimental.pallas import tpu_sc as plsc

_N = 10000
_E = 320000
_DIN = 128
_DH = 256
_DZ = 128
_G = 64
_W = 128

_NC = 2
_NS = 16
_CHUNK = 128
_CPT = 80
_NCH = _NC * _NS * _CPT
_EP = _NCH * _CHUNK
_CPT_COL = _NCH // _NS
_IB = 40
_RPT = 632
_NP = _NS * _RPT

_mesh = plsc.VectorSubcoreMesh(
    core_axis_name="c", subcore_axis_name="s", num_cores=_NC, num_subcores=_NS
)


def _deg_body(dst_hbm, ones_hbm, zeros_hbm, out_hbm, dst_v, ones_v, acc, sem):
    c = lax.axis_index("c")
    s = lax.axis_index("s")
    w = c * _NS + s
    pltpu.sync_copy(ones_hbm, ones_v)
    pltpu.sync_copy(dst_hbm.at[pl.ds(w * _CPT, _CPT)], dst_v)
    pltpu.sync_copy(zeros_hbm, acc.at[pl.ds(s * _RPT, _RPT)])
    plsc.subcore_barrier()

    pltpu.async_copy(ones_v, acc.at[dst_v.at[0]], sem, add=True)

    def body(i, carry):
        d = pltpu.async_copy(ones_v, acc.at[dst_v.at[i + 1]], sem, add=True)
        d.wait()
        return carry

    lax.fori_loop(0, _CPT - 1, body, 0)
    pltpu.make_async_copy(ones_hbm, ones_v, sem).wait()

    plsc.subcore_barrier()
    row0 = s * _RPT
    pltpu.sync_copy(
        acc.at[pl.ds(row0, _RPT)],
        out_hbm.at[pl.ds(c * _NP + row0, _RPT)],
    )


_deg_kernel = functools.partial(
    pl.kernel,
    _deg_body,
    out_type=jax.ShapeDtypeStruct((2 * _NP, _W), jnp.float32),
    mesh=_mesh,
    scratch_types=[
        pltpu.VMEM((_CPT, _CHUNK), jnp.int32),
        pltpu.VMEM((_CHUNK, _W), jnp.float32),
        pltpu.VMEM_SHARED((_NP, _W), jnp.float32),
        pltpu.SemaphoreType.DMA,
    ],
)()


def _edge_body(col_split, table_hbm, src_hbm, dst_hbm, zeros_hbm, out_hbm,
               src_v, dst_v, r0, r1, acc, g0, g1, s0, s1):
    c = lax.axis_index("c")
    s = lax.axis_index("s")
    if col_split:
        n = _CPT_COL
        base = s * n
        plane = c
    else:
        n = _CPT
        base = (c * _NS + s) * n
        plane = 0
    pltpu.sync_copy(zeros_hbm, acc.at[pl.ds(s * _RPT, _RPT)])
    plsc.subcore_barrier()

    def wait_g0():
        pltpu.make_async_copy(table_hbm.at[src_v.at[0]], r0, g0).wait()

    def wait_g1():
        pltpu.make_async_copy(table_hbm.at[src_v.at[0]], r1, g1).wait()

    def wait_s0():
        pltpu.make_async_copy(table_hbm.at[pl.ds(0, _CHUNK)], r0, s0).wait()

    def wait_s1():
        pltpu.make_async_copy(table_hbm.at[pl.ds(0, _CHUNK)], r1, s1).wait()

    def pair(it, carry):
        i0 = 2 * it
        r = lax.rem(i0, _IB)

        @pl.when(r == 0)
        def _():
            @pl.when(i0 > 0)
            def _():
                wait_s1()

            off = pl.multiple_of(base + i0, 8)
            pltpu.sync_copy(src_hbm.at[plane, pl.ds(off, _IB)], src_v)
            pltpu.sync_copy(dst_hbm.at[pl.ds(off, _IB)], dst_v)
            pltpu.async_copy(table_hbm.at[src_v.at[0]], r0, g0)

        wait_g0()
        pltpu.async_copy(r0, acc.at[dst_v.at[r]], s0, add=True)

        @pl.when(r > 0)
        def _():
            wait_s1()

        pltpu.async_copy(table_hbm.at[src_v.at[r + 1]], r1, g1)
        wait_g1()
        pltpu.async_copy(r1, acc.at[dst_v.at[r + 1]], s1, add=True)
        wait_s0()

        @pl.when(r + 2 < _IB)
        def _():
            pltpu.async_copy(table_hbm.at[src_v.at[r + 2]], r0, g0)

        return carry

    lax.fori_loop(0, n // 2, pair, 0)
    wait_s1()

    plsc.subcore_barrier()
    row0 = s * _RPT
    pltpu.sync_copy(
        acc.at[pl.ds(row0, _RPT)],
        out_hbm.at[pl.ds(c * _NP + row0, _RPT)],
    )


def _make_edge_kernel(col_split):
    return functools.partial(
        pl.kernel,
        functools.partial(_edge_body, col_split),
        out_type=jax.ShapeDtypeStruct((2 * _NP, _W), jnp.float32),
        mesh=_mesh,
        scratch_types=[
            pltpu.VMEM((_IB, _CHUNK), jnp.int32),
            pltpu.VMEM((_IB, _CHUNK), jnp.int32),
            pltpu.VMEM((_CHUNK, _W), jnp.float32),
            pltpu.VMEM((_CHUNK, _W), jnp.float32),
            pltpu.VMEM_SHARED((_NP, _W), jnp.float32),
            pltpu.SemaphoreType.DMA,
            pltpu.SemaphoreType.DMA,
            pltpu.SemaphoreType.DMA,
            pltpu.SemaphoreType.DMA,
        ],
    )()


def _edge1_body(table_hbm, src_hbm, dst_hbm, zeros_hbm, out_hbm,
                src_v, dst_v, r0, acc, g0):
    c = lax.axis_index("c")
    s = lax.axis_index("s")
    n = _CPT
    base = (c * _NS + s) * n
    pltpu.sync_copy(zeros_hbm, acc.at[pl.ds(s * _RPT, _RPT)])
    plsc.subcore_barrier()

    def body(i, carry):
        r = lax.rem(i, _IB)

        @pl.when(r == 0)
        def _():
            off = pl.multiple_of(base + i, 8)
            pltpu.sync_copy(src_hbm.at[0, pl.ds(off, _IB)], src_v)
            pltpu.sync_copy(dst_hbm.at[pl.ds(off, _IB)], dst_v)

        pltpu.async_copy(table_hbm.at[src_v.at[r]], r0, g0).wait()
        pltpu.sync_copy(r0, acc.at[dst_v.at[r]], add=True)
        return carry

    lax.fori_loop(0, n, body, 0)

    plsc.subcore_barrier()
    row0 = s * _RPT
    pltpu.sync_copy(
        acc.at[pl.ds(row0, _RPT)],
        out_hbm.at[pl.ds(c * _NP + row0, _RPT)],
    )


_edge_kernel_split = functools.partial(
    pl.kernel,
    _edge1_body,
    out_type=jax.ShapeDtypeStruct((2 * _NP, _W), jnp.float32),
    mesh=_mesh,
    scratch_types=[
        pltpu.VMEM((_IB, _CHUNK), jnp.int32),
        pltpu.VMEM((_IB, _CHUNK), jnp.int32),
        pltpu.VMEM((_CHUNK, _W), jnp.float32),
        pltpu.VMEM_SHARED((_NP, _W), jnp.float32),
        pltpu.SemaphoreType.DMA,
    ],
)()

_edge_kernel_cols = _make_edge_kernel(True)


def _elu(v):
    return jnp.where(v > 0, v, jnp.exp(jnp.minimum(v, 0.0)) - 1.0)


def _tc_prep_body(degp_ref, x_ref, dis_ref, x2_ref):
    deg = degp_ref[0:_N, 0:1] + degp_ref[_NP:_NP + _N, 0:1] + 1.0
    dis = lax.rsqrt(deg)
    dis_ref[...] = dis
    x2_ref[0:_N, :] = x_ref[...] * dis


def _tc_mid_body(s1_ref, x2_ref, dis_ref, w1_ref, b1_ref, h2s_ref):
    dis = dis_ref[...]
    ax = (s1_ref[0:_N, :] + s1_ref[_NP:_NP + _N, :] + x2_ref[0:_N, :]) * dis
    h = _elu(jnp.dot(ax, w1_ref[...], preferred_element_type=jnp.float32)
             + b1_ref[...])
    h2 = h * dis
    h2s_ref[0:_N, :] = h2[:, 0:_DH // 2]
    h2s_ref[_NP:_NP + _N, :] = h2[:, _DH // 2:_DH]


def _tc_head_body(s2_ref, h2s_ref, dis_ref, wmu_ref, bmu_ref, wsg_ref,
                  bsg_ref, batch_ref, zmu_ref, zsg_ref):
    dis = dis_ref[...]
    a = s2_ref[...] + h2s_ref[...]
    ah = jnp.concatenate([a[0:_N, :], a[_NP:_NP + _N, :]], axis=1) * dis
    mu = _elu(jnp.dot(ah, wmu_ref[...], preferred_element_type=jnp.float32)
              + bmu_ref[...])
    sg = _elu(jnp.dot(ah, wsg_ref[...], preferred_element_type=jnp.float32)
              + bsg_ref[...])
    gids = lax.broadcasted_iota(jnp.int32, (_G, _N), 0)
    p = (gids == batch_ref[...]).astype(jnp.float32)
    inv_cnt = 1.0 / jnp.maximum(jnp.sum(p, axis=1, keepdims=True), 1.0)
    zmu_ref[...] = jnp.dot(p, mu, preferred_element_type=jnp.float32) * inv_cnt
    zsg_ref[...] = jnp.dot(p, sg, preferred_element_type=jnp.float32) * inv_cnt


_tc_prep = pl.pallas_call(
    _tc_prep_body,
    out_shape=(
        jax.ShapeDtypeStruct((_N, 1), jnp.float32),
        jax.ShapeDtypeStruct((_NP, _DIN), jnp.float32),
    ),
)

_tc_mid = pl.pallas_call(
    _tc_mid_body,
    out_shape=jax.ShapeDtypeStruct((2 * _NP, _DH // 2), jnp.float32),
)

_tc_head = pl.pallas_call(
    _tc_head_body,
    out_shape=(
        jax.ShapeDtypeStruct((_G, _DZ), jnp.float32),
        jax.ShapeDtypeStruct((_G, _DZ), jnp.float32),
    ),
)


def kernel(x, edge_index, batch, W1, b1, W_mu, b_mu, W_sigma, b_sigma):
    src = jnp.pad(edge_index[0], (0, _EP - _E), constant_values=_N)
    dst = jnp.pad(edge_index[1], (0, _EP - _E), constant_values=_N)
    src2 = src.reshape(_NCH, _CHUNK)
    src3 = jnp.stack([src2, src2 + _NP])
    dst2 = dst.reshape(_NCH, _CHUNK)
    ones_rows = jnp.ones((_CHUNK, _W), jnp.float32)
    zrows = jnp.zeros((_RPT, _W), jnp.float32)

    degp = _deg_kernel(dst2, ones_rows, zrows)
    dis, x2 = _tc_prep(degp, x)
    s1 = _edge_kernel_split(x2, src3, dst2, zrows)
    h2s = _tc_mid(s1, x2, dis, W1, b1.reshape(1, _DH))
    s2 = _edge_kernel_cols(h2s, src3, dst2, zrows)
    z_mu, z_sigma = _tc_head(
        s2, h2s, dis, W_mu, b_mu.reshape(1, _DZ),
        W_sigma, b_sigma.reshape(1, _DZ), batch.reshape(1, _N),
    )
    return (z_mu, z_sigma)


# speedup vs baseline: 1.2626x; 1.2049x over previous
import functools

import jax
import jax.numpy as jnp
from jax import lax
from jax.experimental import pallas as pl
from jax.experimental.pallas import tpu as pltpu
from jax.experimental.pallas import tpu_sc as plsc

_N = 10000
_E = 320000
_DIN = 128
_DH = 256
_DZ = 128
_G = 64
_W = 128

_NC = 2
_NS = 16
_CHUNK = 128
_CPT = 80
_NCH = _NC * _NS * _CPT
_EP = _NCH * _CHUNK
_CPT_COL = _NCH // _NS
_IB = 40
_CHUNK80 = 80
_RPT = 632
_NP = _NS * _RPT

_mesh = plsc.VectorSubcoreMesh(
    core_axis_name="c", subcore_axis_name="s", num_cores=_NC, num_subcores=_NS
)


def _deg_body(dst_hbm, ones_hbm, zeros_hbm, out_hbm, dst_v, ones_v, acc, sem):
    c = lax.axis_index("c")
    s = lax.axis_index("s")
    w = c * _NS + s
    pltpu.sync_copy(ones_hbm, ones_v)
    pltpu.sync_copy(dst_hbm.at[pl.ds(w * _CPT, _CPT)], dst_v)
    pltpu.sync_copy(zeros_hbm, acc.at[pl.ds(s * _RPT, _RPT)])
    plsc.subcore_barrier()

    pltpu.async_copy(ones_v, acc.at[dst_v.at[0]], sem, add=True)

    def body(i, carry):
        d = pltpu.async_copy(ones_v, acc.at[dst_v.at[i + 1]], sem, add=True)
        d.wait()
        return carry

    lax.fori_loop(0, _CPT - 1, body, 0)
    pltpu.make_async_copy(ones_hbm, ones_v, sem).wait()

    plsc.subcore_barrier()
    row0 = s * _RPT
    pltpu.sync_copy(
        acc.at[pl.ds(row0, _RPT)],
        out_hbm.at[pl.ds(c * _NP + row0, _RPT)],
    )


_deg_kernel = functools.partial(
    pl.kernel,
    _deg_body,
    out_type=jax.ShapeDtypeStruct((2 * _NP, _W), jnp.float32),
    mesh=_mesh,
    scratch_types=[
        pltpu.VMEM((_CPT, _CHUNK), jnp.int32),
        pltpu.VMEM((_CHUNK, _W), jnp.float32),
        pltpu.VMEM_SHARED((_NP, _W), jnp.float32),
        pltpu.SemaphoreType.DMA,
    ],
)()


def _edge_body(col_split, table_hbm, src_hbm, dst_hbm, zeros_hbm, out_hbm,
               src_v, dst_v, rows_v, acc, sem):
    c = lax.axis_index("c")
    s = lax.axis_index("s")
    pltpu.sync_copy(zeros_hbm, acc.at[pl.ds(s * _RPT, _RPT)])
    plsc.subcore_barrier()

    if col_split:
        epw = _E // _NS
        ebase = s * epw
    else:
        epw = _E // (_NC * _NS)
        ebase = c * (_E // _NC) + s * epw
    coff = c * _NP

    def body(i, carry):
        off = ebase + i * _CHUNK80
        pltpu.sync_copy(src_hbm.at[pl.ds(off, _CHUNK80)], src_v)
        pltpu.sync_copy(dst_hbm.at[pl.ds(off, _CHUNK80)], dst_v)
        if col_split:
            for j in range(_CHUNK80 // 16):
                src_v[pl.ds(j * 16, 16)] = src_v[pl.ds(j * 16, 16)] + coff
        pltpu.async_copy(table_hbm.at[src_v], rows_v, sem).wait()
        pltpu.sync_copy(rows_v, acc.at[dst_v], add=True)
        return carry

    lax.fori_loop(0, epw // _CHUNK80, body, 0)
    plsc.subcore_barrier()
    row0 = s * _RPT
    pltpu.sync_copy(
        acc.at[pl.ds(row0, _RPT)],
        out_hbm.at[pl.ds(c * _NP + row0, _RPT)],
    )


def _make_edge_kernel(col_split):
    return functools.partial(
        pl.kernel,
        functools.partial(_edge_body, col_split),
        out_type=jax.ShapeDtypeStruct((2 * _NP, _W), jnp.float32),
        mesh=_mesh,
        scratch_types=[
            pltpu.VMEM((_CHUNK80,), jnp.int32),
            pltpu.VMEM((_CHUNK80,), jnp.int32),
            pltpu.VMEM((_CHUNK80, _W), jnp.float32),
            pltpu.VMEM_SHARED((_NP, _W), jnp.float32),
            pltpu.SemaphoreType.DMA,
        ],
    )()


_edge_kernel_split = _make_edge_kernel(False)
_edge_kernel_cols = _make_edge_kernel(True)


def _elu(v):
    return jnp.where(v > 0, v, jnp.exp(jnp.minimum(v, 0.0)) - 1.0)


def _tc_prep_body(degp_ref, x_ref, dis_ref, x2_ref):
    deg = degp_ref[0:_N, 0:1] + degp_ref[_NP:_NP + _N, 0:1] + 1.0
    dis = lax.rsqrt(deg)
    dis_ref[...] = dis
    x2_ref[0:_N, :] = x_ref[...] * dis


def _tc_mid_body(s1_ref, x2_ref, dis_ref, w1_ref, b1_ref, h2s_ref):
    dis = dis_ref[...]
    ax = (s1_ref[0:_N, :] + s1_ref[_NP:_NP + _N, :] + x2_ref[0:_N, :]) * dis
    h = _elu(jnp.dot(ax, w1_ref[...], preferred_element_type=jnp.float32)
             + b1_ref[...])
    h2 = h * dis
    h2s_ref[0:_N, :] = h2[:, 0:_DH // 2]
    h2s_ref[_NP:_NP + _N, :] = h2[:, _DH // 2:_DH]


def _tc_head_body(s2_ref, h2s_ref, dis_ref, wmu_ref, bmu_ref, wsg_ref,
                  bsg_ref, batch_ref, zmu_ref, zsg_ref):
    dis = dis_ref[...]
    a = s2_ref[...] + h2s_ref[...]
    ah = jnp.concatenate([a[0:_N, :], a[_NP:_NP + _N, :]], axis=1) * dis
    mu = _elu(jnp.dot(ah, wmu_ref[...], preferred_element_type=jnp.float32)
              + bmu_ref[...])
    sg = _elu(jnp.dot(ah, wsg_ref[...], preferred_element_type=jnp.float32)
              + bsg_ref[...])
    gids = lax.broadcasted_iota(jnp.int32, (_G, _N), 0)
    p = (gids == batch_ref[...]).astype(jnp.float32)
    inv_cnt = 1.0 / jnp.maximum(jnp.sum(p, axis=1, keepdims=True), 1.0)
    zmu_ref[...] = jnp.dot(p, mu, preferred_element_type=jnp.float32) * inv_cnt
    zsg_ref[...] = jnp.dot(p, sg, preferred_element_type=jnp.float32) * inv_cnt


_tc_prep = pl.pallas_call(
    _tc_prep_body,
    out_shape=(
        jax.ShapeDtypeStruct((_N, 1), jnp.float32),
        jax.ShapeDtypeStruct((_NP, _DIN), jnp.float32),
    ),
)

_tc_mid = pl.pallas_call(
    _tc_mid_body,
    out_shape=jax.ShapeDtypeStruct((2 * _NP, _DH // 2), jnp.float32),
)

_tc_head = pl.pallas_call(
    _tc_head_body,
    out_shape=(
        jax.ShapeDtypeStruct((_G, _DZ), jnp.float32),
        jax.ShapeDtypeStruct((_G, _DZ), jnp.float32),
    ),
)


def kernel(x, edge_index, batch, W1, b1, W_mu, b_mu, W_sigma, b_sigma):
    src = edge_index[0]
    dst = edge_index[1]
    dst2 = jnp.pad(dst, (0, _EP - _E), constant_values=_N).reshape(_NCH, _CHUNK)
    ones_rows = jnp.ones((_CHUNK, _W), jnp.float32)
    zrows = jnp.zeros((_RPT, _W), jnp.float32)

    degp = _deg_kernel(dst2, ones_rows, zrows)
    dis, x2 = _tc_prep(degp, x)
    s1 = _edge_kernel_split(x2, src, dst, zrows)
    h2s = _tc_mid(s1, x2, dis, W1, b1.reshape(1, _DH))
    s2 = _edge_kernel_cols(h2s, src, dst, zrows)
    z_mu, z_sigma = _tc_head(
        s2, h2s, dis, W_mu, b_mu.reshape(1, _DZ),
        W_sigma, b_sigma.reshape(1, _DZ), batch.reshape(1, _N),
    )
    return (z_mu, z_sigma)
